# baseline (speedup 1.0000x reference)
import functools

import jax
import jax.numpy as jnp
from jax import lax
from jax.experimental import pallas as pl
from jax.experimental.pallas import tpu as pltpu
from jax.experimental.pallas import tpu_sc as plsc

N = 4096
D = 512
K = 31
ROW_BLK = 256

NC = 2
NS = 16
NW = NC * NS
RPW = N // NW
NEG = -3.0e38


def _h_kernel(x_ref, w0_ref, b0_ref, w1_ref, b1_ref, h_ref):
    xb = x_ref[...]
    h1 = lax.dot_general(xb, w0_ref[...], (((1,), (1,)), ((), ())),
                         preferred_element_type=jnp.float32)
    h1 = jnp.maximum(h1 + b0_ref[...], 0.0)
    h2 = lax.dot_general(h1, w1_ref[...], (((1,), (1,)), ((), ())),
                         preferred_element_type=jnp.float32)
    h2 = h2 + b1_ref[...]
    ss = jnp.sum(h2 * h2, axis=1, keepdims=True)
    norm = jnp.maximum(jnp.sqrt(ss), 1e-12)
    h_ref[...] = h2 / norm


def _sim_kernel(hb_ref, hall_ref, sim_ref, t0_ref):
    hb = hb_ref[...]
    sim = lax.dot_general(hb, hall_ref[...], (((1,), (1,)), ((), ())),
                          preferred_element_type=jnp.float32)
    sim_ref[...] = sim

    gm = jnp.max(sim.reshape(ROW_BLK, 32, 128), axis=1)

    def body(_, work):
        m = jnp.max(work, axis=1, keepdims=True)
        return jnp.where(work >= m, NEG, work)

    work = lax.fori_loop(0, K - 1, body, gm)
    t0_ref[...] = jnp.max(work, axis=1, keepdims=True)


def _sc_select(sim_hbm, t0_hbm, t_hbm, row_v, cand_v, t0_v, tout_v):
    wid = lax.axis_index("s") * NC + lax.axis_index("c")
    base = wid * RPW
    pltpu.sync_copy(t0_hbm.at[pl.ds(base, RPW)], t0_v.at[pl.ds(0, RPW)])

    def row_body(r, _):
        pltpu.sync_copy(sim_hbm.at[base + r], row_v)
        t0vec = t0_v[pl.ds(r, 16)]
        t0s = jnp.full((16,), t0vec[0])

        def chunk(j, off):
            v = row_v[pl.ds(j * 16, 16)]
            m = v >= t0s
            plsc.store_compressed(cand_v.at[pl.ds(off, 16)], v, mask=m)
            cnt = plsc.all_reduce_population_count(m)
            return off + cnt[0]

        c = lax.fori_loop(0, N // 16, chunk, jnp.int32(0))
        cand_v[pl.ds(c, 16)] = jnp.full((16,), NEG, jnp.float32)
        nv = (c + 15) // 16

        def ext(i, _):
            def mx(j, acc):
                return jnp.maximum(acc, cand_v[pl.ds(j * 16, 16)])

            mv = lax.fori_loop(0, nv, mx, jnp.full((16,), NEG, jnp.float32))
            m = jnp.max(mv)
            ms = jnp.full((16,), m)

            def kill(j, _):
                v = cand_v[pl.ds(j * 16, 16)]
                cand_v[pl.ds(j * 16, 16)] = jnp.where(v == ms, NEG, v)
                return 0

            lax.fori_loop(0, nv, kill, 0)
            return m

        t = lax.fori_loop(0, K, ext, jnp.float32(NEG))
        tout_v[pl.ds(r, 16)] = jnp.full((16,), t)
        return 0

    lax.fori_loop(0, RPW, row_body, 0)
    pltpu.sync_copy(tout_v.at[pl.ds(0, RPW)], t_hbm.at[pl.ds(base, RPW)])


def _mask_kernel(sim_ref, t_ref, out_ref):
    sim = sim_ref[...]
    t = t_ref[...]
    out_ref[...] = jnp.where(sim >= t, jnp.maximum(sim, 0.0), 0.0)


@jax.jit
def kernel(x, W0, b0, W1, b1):
    b0r = b0.reshape(1, D)
    b1r = b1.reshape(1, D)
    h = pl.pallas_call(
        _h_kernel,
        grid=(N // 512,),
        in_specs=[
            pl.BlockSpec((512, D), lambda i: (i, 0)),
            pl.BlockSpec((D, D), lambda i: (0, 0)),
            pl.BlockSpec((1, D), lambda i: (0, 0)),
            pl.BlockSpec((D, D), lambda i: (0, 0)),
            pl.BlockSpec((1, D), lambda i: (0, 0)),
        ],
        out_specs=pl.BlockSpec((512, D), lambda i: (i, 0)),
        out_shape=jax.ShapeDtypeStruct((N, D), jnp.float32),
    )(x, W0, b0r, W1, b1r)

    sim, t0 = pl.pallas_call(
        _sim_kernel,
        grid=(N // ROW_BLK,),
        in_specs=[
            pl.BlockSpec((ROW_BLK, D), lambda i: (i, 0)),
            pl.BlockSpec((N, D), lambda i: (0, 0)),
        ],
        out_specs=[
            pl.BlockSpec((ROW_BLK, N), lambda i: (i, 0)),
            pl.BlockSpec((ROW_BLK, 1), lambda i: (i, 0)),
        ],
        out_shape=[
            jax.ShapeDtypeStruct((N, N), jnp.float32),
            jax.ShapeDtypeStruct((N, 1), jnp.float32),
        ],
    )(h, h)

    sc_select = pl.kernel(
        _sc_select,
        out_type=jax.ShapeDtypeStruct((N,), jnp.float32),
        mesh=plsc.VectorSubcoreMesh(core_axis_name="c", subcore_axis_name="s"),
        compiler_params=pltpu.CompilerParams(needs_layout_passes=False),
        scratch_types=[
            pltpu.VMEM((N,), jnp.float32),
            pltpu.VMEM((N + 16,), jnp.float32),
            pltpu.VMEM((RPW + 16,), jnp.float32),
            pltpu.VMEM((RPW + 16,), jnp.float32),
        ],
    )
    t = sc_select(sim, t0.reshape(N))

    out = pl.pallas_call(
        _mask_kernel,
        grid=(N // ROW_BLK,),
        in_specs=[
            pl.BlockSpec((ROW_BLK, N), lambda i: (i, 0)),
            pl.BlockSpec((ROW_BLK, 1), lambda i: (i, 0)),
        ],
        out_specs=pl.BlockSpec((ROW_BLK, N), lambda i: (i, 0)),
        out_shape=jax.ShapeDtypeStruct((N, N), jnp.float32),
    )(sim, t.reshape(N, 1))
    return out


# speedup vs baseline: 1.5572x; 1.5572x over previous
import functools

import jax
import jax.numpy as jnp
from jax import lax
from jax.experimental import pallas as pl
from jax.experimental.pallas import tpu as pltpu
from jax.experimental.pallas import tpu_sc as plsc

N = 4096
D = 512
K = 31
ROW_BLK = 256

NC = 2
NS = 16
NW = NC * NS
RPW = N // NW
NEG = -3.0e38


def _h_kernel(x_ref, w0_ref, b0_ref, w1_ref, b1_ref, h_ref):
    xb = x_ref[...]
    h1 = lax.dot_general(xb, w0_ref[...], (((1,), (1,)), ((), ())),
                         preferred_element_type=jnp.float32)
    h1 = jnp.maximum(h1 + b0_ref[...], 0.0)
    h2 = lax.dot_general(h1, w1_ref[...], (((1,), (1,)), ((), ())),
                         preferred_element_type=jnp.float32)
    h2 = h2 + b1_ref[...]
    ss = jnp.sum(h2 * h2, axis=1, keepdims=True)
    norm = jnp.maximum(jnp.sqrt(ss), 1e-12)
    h_ref[...] = h2 / norm


def _sim_kernel(hb_ref, hall_ref, sim_ref, t0_ref):
    hb = hb_ref[...]
    sim = lax.dot_general(hb, hall_ref[...], (((1,), (1,)), ((), ())),
                          preferred_element_type=jnp.float32)
    sim_ref[...] = sim

    gm = jnp.max(sim.reshape(ROW_BLK, 32, 128), axis=1)

    def body(_, work):
        m = jnp.max(work, axis=1, keepdims=True)
        return jnp.where(work >= m, NEG, work)

    work = lax.fori_loop(0, K - 1, body, gm)
    t0_ref[...] = jnp.max(work, axis=1, keepdims=True)


GRP = 8
NGRP = RPW // GRP


def _split(a, b):
    rb = jnp.flip(b, 0)
    return jnp.minimum(a, rb), jnp.maximum(a, rb)


def _sc_select(sim_hbm, t0_hbm, t_hbm, buf0, buf1, cand_v, t0_v, tout_v,
               sem0, sem1):
    wid = lax.axis_index("s") * NC + lax.axis_index("c")
    base = wid * RPW
    pltpu.sync_copy(t0_hbm.at[pl.ds(base, RPW)], t0_v.at[pl.ds(0, RPW)])

    def start(g, buf, sem):
        pltpu.async_copy(sim_hbm.at[pl.ds(base + g * GRP, GRP)], buf, sem)

    def wait(buf, sem):
        pltpu.make_async_copy(sim_hbm.at[pl.ds(base, GRP)], buf, sem).wait()

    def do_row(r, buf, ri):
        t0vec = t0_v[pl.ds(r, 16)]
        t0s = jnp.full((16,), t0vec[0])

        def chunk(j, off):
            v = buf[ri, pl.ds(j * 16, 16)]
            m = v >= t0s
            plsc.store_compressed(cand_v.at[pl.ds(off, 16)], v, mask=m)
            cnt = plsc.all_reduce_population_count(m)
            return off + cnt[0]

        c = lax.fori_loop(0, N // 16, chunk, jnp.int32(0), unroll=8)
        cand_v[pl.ds(c, 16)] = jnp.full((16,), NEG, jnp.float32)
        nv = (c + 15) // 16

        a0 = jnp.sort(cand_v[pl.ds(0, 16)])
        b0 = jnp.sort(cand_v[pl.ds(16, 16)])
        lo, hi = _split(a0, b0)
        carry = (jnp.sort(hi), jnp.sort(lo))

        def merge(j, ab):
            a, b = ab
            sv = jnp.sort(cand_v[pl.ds(j * 16, 16)])
            lo1, hi1 = _split(a, sv)
            a2 = jnp.sort(hi1)
            lo1s = jnp.sort(lo1)
            _, hi2 = _split(b, lo1s)
            return (a2, jnp.sort(hi2))

        a, b = lax.fori_loop(2, nv, merge, carry)
        t = b[1]
        tout_v[pl.ds(r, 16)] = jnp.full((16,), t)

    start(0, buf0, sem0)

    def pair_body(g, _):
        wait(buf0, sem0)
        start(2 * g + 1, buf1, sem1)

        def rows0(i, _):
            do_row((2 * g) * GRP + i, buf0, i)
            return 0

        lax.fori_loop(0, GRP, rows0, 0)
        wait(buf1, sem1)

        @pl.when(g < NGRP // 2 - 1)
        def _():
            start(2 * g + 2, buf0, sem0)

        def rows1(i, _):
            do_row((2 * g + 1) * GRP + i, buf1, i)
            return 0

        lax.fori_loop(0, GRP, rows1, 0)
        return 0

    lax.fori_loop(0, NGRP // 2, pair_body, 0)
    pltpu.sync_copy(tout_v.at[pl.ds(0, RPW)], t_hbm.at[pl.ds(base, RPW)])


def _mask_kernel(sim_ref, t_ref, out_ref):
    sim = sim_ref[...]
    t = t_ref[...]
    out_ref[...] = jnp.where(sim >= t, jnp.maximum(sim, 0.0), 0.0)


@jax.jit
def kernel(x, W0, b0, W1, b1):
    b0r = b0.reshape(1, D)
    b1r = b1.reshape(1, D)
    h = pl.pallas_call(
        _h_kernel,
        grid=(N // 512,),
        in_specs=[
            pl.BlockSpec((512, D), lambda i: (i, 0)),
            pl.BlockSpec((D, D), lambda i: (0, 0)),
            pl.BlockSpec((1, D), lambda i: (0, 0)),
            pl.BlockSpec((D, D), lambda i: (0, 0)),
            pl.BlockSpec((1, D), lambda i: (0, 0)),
        ],
        out_specs=pl.BlockSpec((512, D), lambda i: (i, 0)),
        out_shape=jax.ShapeDtypeStruct((N, D), jnp.float32),
    )(x, W0, b0r, W1, b1r)

    sim, t0 = pl.pallas_call(
        _sim_kernel,
        grid=(N // ROW_BLK,),
        in_specs=[
            pl.BlockSpec((ROW_BLK, D), lambda i: (i, 0)),
            pl.BlockSpec((N, D), lambda i: (0, 0)),
        ],
        out_specs=[
            pl.BlockSpec((ROW_BLK, N), lambda i: (i, 0)),
            pl.BlockSpec((ROW_BLK, 1), lambda i: (i, 0)),
        ],
        out_shape=[
            jax.ShapeDtypeStruct((N, N), jnp.float32),
            jax.ShapeDtypeStruct((N, 1), jnp.float32),
        ],
    )(h, h)

    sc_select = pl.kernel(
        _sc_select,
        out_type=jax.ShapeDtypeStruct((N,), jnp.float32),
        mesh=plsc.VectorSubcoreMesh(core_axis_name="c", subcore_axis_name="s"),
        compiler_params=pltpu.CompilerParams(needs_layout_passes=False),
        scratch_types=[
            pltpu.VMEM((GRP, N), jnp.float32),
            pltpu.VMEM((GRP, N), jnp.float32),
            pltpu.VMEM((N + 16,), jnp.float32),
            pltpu.VMEM((RPW + 16,), jnp.float32),
            pltpu.VMEM((RPW + 16,), jnp.float32),
            pltpu.SemaphoreType.DMA,
            pltpu.SemaphoreType.DMA,
        ],
    )
    t = sc_select(sim, t0.reshape(N))

    out = pl.pallas_call(
        _mask_kernel,
        grid=(N // ROW_BLK,),
        in_specs=[
            pl.BlockSpec((ROW_BLK, N), lambda i: (i, 0)),
            pl.BlockSpec((ROW_BLK, 1), lambda i: (i, 0)),
        ],
        out_specs=pl.BlockSpec((ROW_BLK, N), lambda i: (i, 0)),
        out_shape=jax.ShapeDtypeStruct((N, N), jnp.float32),
    )(sim, t.reshape(N, 1))
    return out


# speedup vs baseline: 1.9036x; 1.2224x over previous
import functools

import jax
import jax.numpy as jnp
from jax import lax
from jax.experimental import pallas as pl
from jax.experimental.pallas import tpu as pltpu
from jax.experimental.pallas import tpu_sc as plsc

N = 4096
D = 512
K = 31
ROW_BLK = 256

NC = 2
NS = 16
NW = NC * NS
RPW = N // NW
NEG = -3.0e38


def _h_kernel(x_ref, w0_ref, b0_ref, w1_ref, b1_ref, h_ref):
    xb = x_ref[...]
    h1 = lax.dot_general(xb, w0_ref[...], (((1,), (1,)), ((), ())),
                         preferred_element_type=jnp.float32)
    h1 = jnp.maximum(h1 + b0_ref[...], 0.0)
    h2 = lax.dot_general(h1, w1_ref[...], (((1,), (1,)), ((), ())),
                         preferred_element_type=jnp.float32)
    h2 = h2 + b1_ref[...]
    ss = jnp.sum(h2 * h2, axis=1, keepdims=True)
    norm = jnp.maximum(jnp.sqrt(ss), 1e-12)
    h_ref[...] = h2 / norm


def _sim_kernel(hb_ref, hall_ref, sim_ref, t0_ref):
    hb = hb_ref[...]
    sim = lax.dot_general(hb, hall_ref[...], (((1,), (1,)), ((), ())),
                          preferred_element_type=jnp.float32)
    sim_ref[...] = sim

    gm = jnp.max(sim.reshape(ROW_BLK, 32, 128), axis=1)

    def body(_, work):
        m = jnp.max(work, axis=1, keepdims=True)
        return jnp.where(work >= m, NEG, work)

    work = lax.fori_loop(0, K - 1, body, gm)
    t0_ref[...] = jnp.max(work, axis=1, keepdims=True)


GRP = 8
NGRP = RPW // GRP


def _split(a, b):
    rb = jnp.flip(b, 0)
    return jnp.minimum(a, rb), jnp.maximum(a, rb)


def _sc_select(sim_hbm, t0_hbm, t_hbm, buf0, buf1, cand_v, cand2_v, t0_v,
               tout_v, sem0, sem1):
    wid = lax.axis_index("s") * NC + lax.axis_index("c")
    base = wid * RPW
    pltpu.sync_copy(t0_hbm.at[pl.ds(base, RPW)], t0_v.at[pl.ds(0, RPW)])

    def start(g, buf, sem):
        pltpu.async_copy(sim_hbm.at[pl.ds(base + g * GRP, GRP)], buf, sem)

    def wait(buf, sem):
        pltpu.make_async_copy(sim_hbm.at[pl.ds(base, GRP)], buf, sem).wait()

    def topk_of_cand(cand, c):
        nv = (c + 15) // 16
        a0 = jnp.sort(cand[pl.ds(0, 16)])
        b0 = jnp.sort(cand[pl.ds(16, 16)])
        lo, hi = _split(a0, b0)
        carry = (jnp.sort(hi), jnp.sort(lo))

        def merge(j, ab):
            a, b = ab
            sv = jnp.sort(cand[pl.ds(j * 16, 16)])
            lo1, hi1 = _split(a, sv)
            a2 = jnp.sort(hi1)
            lo1s = jnp.sort(lo1)
            _, hi2 = _split(b, lo1s)
            return (a2, jnp.sort(hi2))

        a, b = lax.fori_loop(2, nv, merge, carry)
        return b[1]

    def do_rows(r, buf, ri):
        t0vec = t0_v[pl.ds(r, 16)]
        t0s0 = jnp.full((16,), t0vec[0])
        t0s1 = jnp.full((16,), t0vec[1])

        def chunk(j, offs):
            o0, o1 = offs
            v0 = buf[ri, pl.ds(j * 16, 16)]
            v1 = buf[ri + 1, pl.ds(j * 16, 16)]
            m0 = v0 >= t0s0
            m1 = v1 >= t0s1
            plsc.store_compressed(cand_v.at[pl.ds(o0, 16)], v0, mask=m0)
            plsc.store_compressed(cand2_v.at[pl.ds(o1, 16)], v1, mask=m1)
            c0 = plsc.all_reduce_population_count(m0)
            c1 = plsc.all_reduce_population_count(m1)
            return (o0 + c0[0], o1 + c1[0])

        c0, c1 = lax.fori_loop(0, N // 16, chunk,
                               (jnp.int32(0), jnp.int32(0)), unroll=8)
        cand_v[pl.ds(c0, 16)] = jnp.full((16,), NEG, jnp.float32)
        cand2_v[pl.ds(c1, 16)] = jnp.full((16,), NEG, jnp.float32)
        t0_ = topk_of_cand(cand_v, c0)
        t1_ = topk_of_cand(cand2_v, c1)
        tout_v[pl.ds(r, 16)] = jnp.full((16,), t0_)
        tout_v[pl.ds(r + 1, 16)] = jnp.full((16,), t1_)

    start(0, buf0, sem0)

    def pair_body(g, _):
        wait(buf0, sem0)
        start(2 * g + 1, buf1, sem1)

        def rows0(i, _):
            do_rows((2 * g) * GRP + 2 * i, buf0, 2 * i)
            return 0

        lax.fori_loop(0, GRP // 2, rows0, 0)
        wait(buf1, sem1)

        @pl.when(g < NGRP // 2 - 1)
        def _():
            start(2 * g + 2, buf0, sem0)

        def rows1(i, _):
            do_rows((2 * g + 1) * GRP + 2 * i, buf1, 2 * i)
            return 0

        lax.fori_loop(0, GRP // 2, rows1, 0)
        return 0

    lax.fori_loop(0, NGRP // 2, pair_body, 0)
    pltpu.sync_copy(tout_v.at[pl.ds(0, RPW)], t_hbm.at[pl.ds(base, RPW)])


def _mask_kernel(sim_ref, t_ref, out_ref):
    sim = sim_ref[...]
    t = t_ref[...]
    out_ref[...] = jnp.where(sim >= t, jnp.maximum(sim, 0.0), 0.0)


@jax.jit
def kernel(x, W0, b0, W1, b1):
    b0r = b0.reshape(1, D)
    b1r = b1.reshape(1, D)
    h = pl.pallas_call(
        _h_kernel,
        grid=(N // 512,),
        in_specs=[
            pl.BlockSpec((512, D), lambda i: (i, 0)),
            pl.BlockSpec((D, D), lambda i: (0, 0)),
            pl.BlockSpec((1, D), lambda i: (0, 0)),
            pl.BlockSpec((D, D), lambda i: (0, 0)),
            pl.BlockSpec((1, D), lambda i: (0, 0)),
        ],
        out_specs=pl.BlockSpec((512, D), lambda i: (i, 0)),
        out_shape=jax.ShapeDtypeStruct((N, D), jnp.float32),
    )(x, W0, b0r, W1, b1r)

    sim, t0 = pl.pallas_call(
        _sim_kernel,
        grid=(N // ROW_BLK,),
        in_specs=[
            pl.BlockSpec((ROW_BLK, D), lambda i: (i, 0)),
            pl.BlockSpec((N, D), lambda i: (0, 0)),
        ],
        out_specs=[
            pl.BlockSpec((ROW_BLK, N), lambda i: (i, 0)),
            pl.BlockSpec((ROW_BLK, 1), lambda i: (i, 0)),
        ],
        out_shape=[
            jax.ShapeDtypeStruct((N, N), jnp.float32),
            jax.ShapeDtypeStruct((N, 1), jnp.float32),
        ],
    )(h, h)

    sc_select = pl.kernel(
        _sc_select,
        out_type=jax.ShapeDtypeStruct((N,), jnp.float32),
        mesh=plsc.VectorSubcoreMesh(core_axis_name="c", subcore_axis_name="s"),
        compiler_params=pltpu.CompilerParams(needs_layout_passes=False),
        scratch_types=[
            pltpu.VMEM((GRP, N), jnp.float32),
            pltpu.VMEM((GRP, N), jnp.float32),
            pltpu.VMEM((N + 16,), jnp.float32),
            pltpu.VMEM((N + 16,), jnp.float32),
            pltpu.VMEM((RPW + 16,), jnp.float32),
            pltpu.VMEM((RPW + 16,), jnp.float32),
            pltpu.SemaphoreType.DMA,
            pltpu.SemaphoreType.DMA,
        ],
    )
    t = sc_select(sim, t0.reshape(N))

    out = pl.pallas_call(
        _mask_kernel,
        grid=(N // ROW_BLK,),
        in_specs=[
            pl.BlockSpec((ROW_BLK, N), lambda i: (i, 0)),
            pl.BlockSpec((ROW_BLK, 1), lambda i: (i, 0)),
        ],
        out_specs=pl.BlockSpec((ROW_BLK, N), lambda i: (i, 0)),
        out_shape=jax.ShapeDtypeStruct((N, N), jnp.float32),
    )(sim, t.reshape(N, 1))
    return out
